# baseline (speedup 1.0000x reference)
import jax
import jax.numpy as jnp
from jax import lax
from jax.experimental import pallas as pl
from jax.experimental.pallas import tpu as pltpu
from jax.experimental.pallas import tpu_sc as plsc

DEP_VOCAB = 50
EMBED_DIM = 64
B, S = 8, 256
N = B * S * S
NC, NS = 2, 16
NW = NC * NS
NROWS = B * S
ROWS_PER_W = NROWS // NW
ROW_WORDS = EMBED_DIM * S
CHUNK_ROWS = 2
CHUNK = CHUNK_ROWS * S
NCHUNK = ROWS_PER_W // CHUNK_ROWS
LANES = 16


def _sc_body(idx_hbm, adj_hbm, tab_hbm, out_hbm, tab_v, idx_v, adj_v, out_v):
    wid = lax.axis_index("s") * NC + lax.axis_index("c")
    row0 = wid * ROWS_PER_W
    pltpu.sync_copy(tab_hbm, tab_v)

    def chunk_body(ci, carry):
        rbase = row0 + ci * CHUNK_ROWS
        pltpu.sync_copy(idx_hbm.at[pl.ds(rbase * S, CHUNK)], idx_v)
        pltpu.sync_copy(adj_hbm.at[pl.ds(rbase * S, CHUNK)], adj_v)

        def group_body(g, c2):
            off = g * LANES
            idx16 = idx_v[pl.ds(off, LANES)]
            adj16 = adj_v[pl.ds(off, LANES)]
            idx64 = idx16 * EMBED_DIM
            obase = ((g & ~15) << 10) | ((g & 15) << 4)

            @plsc.parallel_loop(0, EMBED_DIM, unroll=8)
            def d_loop(d):
                r = plsc.load_gather(tab_v, [idx64 + d])
                out_v[pl.ds(obase + d * S, LANES)] = r * adj16

            return c2

        lax.fori_loop(0, CHUNK // LANES, group_body, 0)
        pltpu.sync_copy(out_v, out_hbm.at[pl.ds(rbase * ROW_WORDS,
                                                CHUNK_ROWS * ROW_WORDS)])
        return carry

    lax.fori_loop(0, NCHUNK, chunk_body, 0)


@jax.jit
def _sc_call(idx, adjf, tab):
    mesh = plsc.VectorSubcoreMesh(core_axis_name="c", subcore_axis_name="s",
                                  num_cores=NC, num_subcores=NS)
    fn = pl.kernel(
        _sc_body,
        out_type=jax.ShapeDtypeStruct((N * EMBED_DIM,), jnp.float32),
        mesh=mesh,
        compiler_params=pltpu.CompilerParams(needs_layout_passes=False),
        scratch_types=[
            pltpu.VMEM((DEP_VOCAB * EMBED_DIM,), jnp.float32),
            pltpu.VMEM((CHUNK,), jnp.int32),
            pltpu.VMEM((CHUNK,), jnp.float32),
            pltpu.VMEM((CHUNK_ROWS * ROW_WORDS,), jnp.float32),
        ],
    )
    return fn(idx, adjf, tab)


def kernel(adj_matrix, dep_rel_matrix, dep_table):
    idx = dep_rel_matrix.reshape(-1).astype(jnp.int32)
    adjf = adj_matrix.reshape(-1).astype(jnp.float32)
    tab = dep_table.reshape(-1).astype(jnp.float32)
    out = _sc_call(idx, adjf, tab)
    return out.reshape(B, S, EMBED_DIM, S).transpose(0, 1, 3, 2)


# speedup vs baseline: 2.0813x; 2.0813x over previous
import jax
import jax.numpy as jnp
from jax import lax
from jax.experimental import pallas as pl
from jax.experimental.pallas import tpu as pltpu
from jax.experimental.pallas import tpu_sc as plsc

DEP_VOCAB = 50
EMBED_DIM = 64
B, S = 8, 256
N = B * S * S
NC, NS = 2, 16
NW = NC * NS
NROWS = B * S
ROWS_PER_W = NROWS // NW
ROW_WORDS = EMBED_DIM * S
CHUNK_ROWS = 2
CHUNK = CHUNK_ROWS * S
NCHUNK = ROWS_PER_W // CHUNK_ROWS
LANES = 16


def _sc_body(idx_hbm, adj_hbm, tab_hbm, out_hbm, tab_v, idx_v, adj_v, out_v):
    wid = lax.axis_index("s") * NC + lax.axis_index("c")
    row0 = wid * ROWS_PER_W
    pltpu.sync_copy(tab_hbm, tab_v)

    def chunk_body(ci, carry):
        rbase = row0 + ci * CHUNK_ROWS
        pltpu.sync_copy(idx_hbm.at[pl.ds(rbase * S, CHUNK)], idx_v)
        pltpu.sync_copy(adj_hbm.at[pl.ds(rbase * S, CHUNK)], adj_v)

        def group_body(g, c2):
            off = g * LANES
            idx16 = idx_v[pl.ds(off, LANES)]
            adj16 = adj_v[pl.ds(off, LANES)]
            idx64 = idx16 * EMBED_DIM
            obase = ((g & ~15) << 10) | ((g & 15) << 4)
            iota = lax.iota(jnp.int32, LANES)
            jb = iota + obase

            @plsc.parallel_loop(0, LANES, unroll=4)
            def s_loop(s):
                ps = (iota + s) & (LANES - 1)
                idxp = idx64 + ps
                sb = jb + (ps << 8)
                for c in range(4):
                    r = plsc.load_gather(tab_v, [idxp + c * LANES])
                    plsc.store_scatter(out_v, [sb + c * (LANES * S)],
                                       r * adj16)

            return c2

        lax.fori_loop(0, CHUNK // LANES, group_body, 0)
        pltpu.sync_copy(out_v, out_hbm.at[pl.ds(rbase * ROW_WORDS,
                                                CHUNK_ROWS * ROW_WORDS)])
        return carry

    lax.fori_loop(0, NCHUNK, chunk_body, 0)


@jax.jit
def _sc_call(idx, adjf, tab):
    mesh = plsc.VectorSubcoreMesh(core_axis_name="c", subcore_axis_name="s",
                                  num_cores=NC, num_subcores=NS)
    fn = pl.kernel(
        _sc_body,
        out_type=jax.ShapeDtypeStruct((N * EMBED_DIM,), jnp.float32),
        mesh=mesh,
        compiler_params=pltpu.CompilerParams(needs_layout_passes=False),
        scratch_types=[
            pltpu.VMEM((DEP_VOCAB * EMBED_DIM,), jnp.float32),
            pltpu.VMEM((CHUNK,), jnp.int32),
            pltpu.VMEM((CHUNK,), jnp.float32),
            pltpu.VMEM((CHUNK_ROWS * ROW_WORDS,), jnp.float32),
        ],
    )
    return fn(idx, adjf, tab)


def kernel(adj_matrix, dep_rel_matrix, dep_table):
    idx = dep_rel_matrix.reshape(-1).astype(jnp.int32)
    adjf = adj_matrix.reshape(-1).astype(jnp.float32)
    tab = dep_table.reshape(-1).astype(jnp.float32)
    out = _sc_call(idx, adjf, tab)
    return out.reshape(B, S, EMBED_DIM, S).transpose(0, 1, 3, 2)


# speedup vs baseline: 3.3359x; 1.6028x over previous
import jax
import jax.numpy as jnp
from jax import lax
from jax.experimental import pallas as pl
from jax.experimental.pallas import tpu as pltpu
from jax.experimental.pallas import tpu_sc as plsc

DEP_VOCAB = 50
EMBED_DIM = 64
B, S = 8, 256
N = B * S * S
NC, NS = 2, 16
NW = NC * NS
NROWS = B * S
ROWS_PER_W = NROWS // NW
CHUNK_ROWS = 2
CHUNK = CHUNK_ROWS * S
NCHUNK = ROWS_PER_W // CHUNK_ROWS
LANES = 16


def _sc_body(idx_hbm, adj_hbm, tab_hbm, out_hbm, tab_v, idx_v, adj_v, out_v):
    wid = lax.axis_index("s") * NC + lax.axis_index("c")
    row0 = wid * ROWS_PER_W
    pltpu.sync_copy(tab_hbm, tab_v)

    def chunk_body(ci, carry):
        rbase = row0 + ci * CHUNK_ROWS
        pltpu.sync_copy(idx_hbm.at[pl.ds(rbase * S, CHUNK)], idx_v)
        pltpu.sync_copy(adj_hbm.at[pl.ds(rbase * S, CHUNK)], adj_v)

        def group_body(g, c2):
            off = g * LANES
            idx16 = idx_v[pl.ds(off, LANES)]
            adj16 = adj_v[pl.ds(off, LANES)]
            idx64 = idx16 * EMBED_DIM
            iota = lax.iota(jnp.int32, LANES)
            rvec = jnp.full((LANES,), g >> 4, dtype=jnp.int32)
            jvec = iota + ((g & 15) << 4)

            @plsc.parallel_loop(0, LANES, unroll=4)
            def s_loop(s):
                ps = (iota + s) & (LANES - 1)
                idxp = idx64 + ps
                for c in range(4):
                    r = plsc.load_gather(tab_v, [idxp + c * LANES])
                    plsc.store_scatter(out_v, [rvec, ps + c * LANES, jvec],
                                       r * adj16)

            return c2

        lax.fori_loop(0, CHUNK // LANES, group_body, 0)
        b = rbase >> 8
        i0 = rbase & (S - 1)
        pltpu.sync_copy(out_v, out_hbm.at[b, pl.ds(i0, CHUNK_ROWS)])
        return carry

    lax.fori_loop(0, NCHUNK, chunk_body, 0)


@jax.jit
def _sc_call(idx, adjf, tab):
    mesh = plsc.VectorSubcoreMesh(core_axis_name="c", subcore_axis_name="s",
                                  num_cores=NC, num_subcores=NS)
    fn = pl.kernel(
        _sc_body,
        out_type=jax.ShapeDtypeStruct((B, S, EMBED_DIM, S), jnp.float32),
        mesh=mesh,
        compiler_params=pltpu.CompilerParams(needs_layout_passes=False),
        scratch_types=[
            pltpu.VMEM((DEP_VOCAB * EMBED_DIM,), jnp.float32),
            pltpu.VMEM((CHUNK,), jnp.int32),
            pltpu.VMEM((CHUNK,), jnp.float32),
            pltpu.VMEM((CHUNK_ROWS, EMBED_DIM, S), jnp.float32),
        ],
    )
    return fn(idx, adjf, tab)


def kernel(adj_matrix, dep_rel_matrix, dep_table):
    idx = dep_rel_matrix.reshape(-1).astype(jnp.int32)
    adjf = adj_matrix.reshape(-1).astype(jnp.float32)
    tab = dep_table.reshape(-1).astype(jnp.float32)
    out = _sc_call(idx, adjf, tab)
    return out.transpose(0, 1, 3, 2)


# speedup vs baseline: 5.3265x; 1.5967x over previous
import jax
import jax.numpy as jnp
from jax import lax
from jax.experimental import pallas as pl
from jax.experimental.pallas import tpu as pltpu
from jax.experimental.pallas import tpu_sc as plsc

DEP_VOCAB = 50
EMBED_DIM = 64
B, S = 8, 256
N = B * S * S
NC, NS = 2, 16
NW = NC * NS
NROWS = B * S
ROWS_PER_W = NROWS // NW
CHUNK_ROWS = 2
CHUNK = CHUNK_ROWS * S
NCHUNK = ROWS_PER_W // CHUNK_ROWS
NSUPER = NCHUNK // 2
LANES = 16


def _sc_body(idx_hbm, adj_hbm, tab_hbm, out_hbm, tab_v, idx_vs, adj_vs,
             out_vs, in_sems, out_sems):
    wid = lax.axis_index("s") * NC + lax.axis_index("c")
    row0 = wid * ROWS_PER_W
    pltpu.sync_copy(tab_hbm, tab_v)

    def in_descs(ci, buf):
        rbase = jnp.minimum(row0 + ci * CHUNK_ROWS, NROWS - CHUNK_ROWS)
        return (
            pltpu.make_async_copy(idx_hbm.at[pl.ds(rbase * S, CHUNK)],
                                  idx_vs[buf], in_sems[buf]),
            pltpu.make_async_copy(adj_hbm.at[pl.ds(rbase * S, CHUNK)],
                                  adj_vs[buf], in_sems[buf]),
        )

    def out_desc(ci, buf):
        rbase = row0 + ci * CHUNK_ROWS
        return pltpu.make_async_copy(
            out_vs[buf],
            out_hbm.at[rbase >> 8, pl.ds(rbase & (S - 1), CHUNK_ROWS)],
            out_sems[buf])

    def compute(buf):
        ivb = idx_vs[buf]
        avb = adj_vs[buf]
        ovb = out_vs[buf]

        def group_body(g, c2):
            off = g * LANES
            idx16 = ivb[pl.ds(off, LANES)]
            adj16 = avb[pl.ds(off, LANES)]
            idx64 = idx16 * EMBED_DIM
            iota = lax.iota(jnp.int32, LANES)
            rvec = jnp.full((LANES,), g >> 4, dtype=jnp.int32)
            jvec = iota + ((g & 15) << 4)

            @plsc.parallel_loop(0, LANES, unroll=4)
            def s_loop(s):
                ps = (iota + s) & (LANES - 1)
                idxp = idx64 + ps
                for c in range(4):
                    r = plsc.load_gather(tab_v, [idxp + c * LANES])
                    plsc.store_scatter(ovb, [rvec, ps + c * LANES, jvec],
                                       r * adj16)

            return c2

        lax.fori_loop(0, CHUNK // LANES, group_body, 0)

    for d in in_descs(0, 0) + in_descs(1, 1):
        d.start()

    def super_body(sc, carry):
        for buf in range(2):
            ci = sc * 2 + buf
            for d in in_descs(ci, buf):
                d.wait()
            for d in in_descs(ci + 2, buf):
                d.start()

            @pl.when(sc >= 1)
            def _():
                out_desc(ci - 2, buf).wait()

            compute(buf)
            out_desc(ci, buf).start()
        return carry

    lax.fori_loop(0, NSUPER, super_body, 0)
    out_desc(NCHUNK - 2, 0).wait()
    out_desc(NCHUNK - 1, 1).wait()
    for buf in range(2):
        for d in in_descs(NCHUNK + buf, buf):
            d.wait()


@jax.jit
def _sc_call(idx, adjf, tab):
    mesh = plsc.VectorSubcoreMesh(core_axis_name="c", subcore_axis_name="s",
                                  num_cores=NC, num_subcores=NS)
    fn = pl.kernel(
        _sc_body,
        out_type=jax.ShapeDtypeStruct((B, S, EMBED_DIM, S), jnp.float32),
        mesh=mesh,
        compiler_params=pltpu.CompilerParams(needs_layout_passes=False),
        scratch_types=[
            pltpu.VMEM((DEP_VOCAB * EMBED_DIM,), jnp.float32),
            [pltpu.VMEM((CHUNK,), jnp.int32),
             pltpu.VMEM((CHUNK,), jnp.int32)],
            [pltpu.VMEM((CHUNK,), jnp.float32),
             pltpu.VMEM((CHUNK,), jnp.float32)],
            [pltpu.VMEM((CHUNK_ROWS, EMBED_DIM, S), jnp.float32),
             pltpu.VMEM((CHUNK_ROWS, EMBED_DIM, S), jnp.float32)],
            [pltpu.SemaphoreType.DMA, pltpu.SemaphoreType.DMA],
            [pltpu.SemaphoreType.DMA, pltpu.SemaphoreType.DMA],
        ],
    )
    return fn(idx, adjf, tab)


def kernel(adj_matrix, dep_rel_matrix, dep_table):
    idx = dep_rel_matrix.reshape(-1).astype(jnp.int32)
    adjf = adj_matrix.reshape(-1).astype(jnp.float32)
    tab = dep_table.reshape(-1).astype(jnp.float32)
    out = _sc_call(idx, adjf, tab)
    return out.transpose(0, 1, 3, 2)


# speedup vs baseline: 5.3638x; 1.0070x over previous
import jax
import jax.numpy as jnp
from jax import lax
from jax.experimental import pallas as pl
from jax.experimental.pallas import tpu as pltpu
from jax.experimental.pallas import tpu_sc as plsc

DEP_VOCAB = 50
EMBED_DIM = 64
B, S = 8, 256
N = B * S * S
NC, NS = 2, 16
NW = NC * NS
NROWS = B * S
ROWS_PER_W = NROWS // NW
CHUNK_ROWS = 2
CHUNK = CHUNK_ROWS * S
NCHUNK = ROWS_PER_W // CHUNK_ROWS
NSUPER = NCHUNK // 2
LANES = 16


def _sc_body(idx_hbm, adj_hbm, tab_hbm, out_hbm, tab_v, idx_vs, adj_vs,
             out_vs, in_sems, out_sems):
    wid = lax.axis_index("s") * NC + lax.axis_index("c")
    row0 = wid * ROWS_PER_W
    pltpu.sync_copy(tab_hbm, tab_v)

    def in_descs(ci, buf):
        rbase = jnp.minimum(row0 + ci * CHUNK_ROWS, NROWS - CHUNK_ROWS)
        return (
            pltpu.make_async_copy(idx_hbm.at[pl.ds(rbase * S, CHUNK)],
                                  idx_vs[buf], in_sems[buf]),
            pltpu.make_async_copy(adj_hbm.at[pl.ds(rbase * S, CHUNK)],
                                  adj_vs[buf], in_sems[buf]),
        )

    def out_desc(ci, buf):
        rbase = row0 + ci * CHUNK_ROWS
        return pltpu.make_async_copy(
            out_vs[buf],
            out_hbm.at[rbase >> 8, pl.ds(rbase & (S - 1), CHUNK_ROWS)],
            out_sems[buf])

    def compute(buf):
        ivb = idx_vs[buf]
        avb = adj_vs[buf]
        ovb = out_vs[buf]

        def group_body(g, c2):
            off = g * LANES
            idx16 = ivb[pl.ds(off, LANES)]
            adj16 = avb[pl.ds(off, LANES)]
            idx64 = idx16 * EMBED_DIM
            iota = lax.iota(jnp.int32, LANES)
            rvec = jnp.full((LANES,), g >> 4, dtype=jnp.int32)
            jvec = iota + ((g & 15) << 4)

            @plsc.parallel_loop(0, LANES, unroll=4)
            def s_loop(s):
                ps = (iota + s) & (LANES - 1)
                idxp = idx64 + ps
                for c in range(4):
                    r = plsc.load_gather(tab_v, [idxp + c * LANES])
                    plsc.store_scatter(ovb, [rvec, ps + c * LANES, jvec],
                                       r * adj16)

            return c2

        lax.fori_loop(0, CHUNK // LANES, group_body, 0)

    for d in in_descs(0, 0) + in_descs(1, 1):
        d.start()

    def super_body(sc, carry):
        for buf in range(2):
            ci = sc * 2 + buf
            for d in in_descs(ci, buf):
                d.wait()

            @pl.when(sc >= 1)
            def _():
                out_desc(ci - 2, buf).wait()

            compute(buf)
            for d in in_descs(ci + 2, buf):
                d.start()
            out_desc(ci, buf).start()
        return carry

    lax.fori_loop(0, NSUPER, super_body, 0)
    out_desc(NCHUNK - 2, 0).wait()
    out_desc(NCHUNK - 1, 1).wait()
    for buf in range(2):
        for d in in_descs(NCHUNK + buf, buf):
            d.wait()


@jax.jit
def _sc_call(idx, adjf, tab):
    mesh = plsc.VectorSubcoreMesh(core_axis_name="c", subcore_axis_name="s",
                                  num_cores=NC, num_subcores=NS)
    fn = pl.kernel(
        _sc_body,
        out_type=jax.ShapeDtypeStruct((B, S, EMBED_DIM, S), jnp.float32),
        mesh=mesh,
        compiler_params=pltpu.CompilerParams(needs_layout_passes=False),
        scratch_types=[
            pltpu.VMEM((DEP_VOCAB * EMBED_DIM,), jnp.float32),
            [pltpu.VMEM((CHUNK,), jnp.int32),
             pltpu.VMEM((CHUNK,), jnp.int32)],
            [pltpu.VMEM((CHUNK,), jnp.float32),
             pltpu.VMEM((CHUNK,), jnp.float32)],
            [pltpu.VMEM((CHUNK_ROWS, EMBED_DIM, S), jnp.float32),
             pltpu.VMEM((CHUNK_ROWS, EMBED_DIM, S), jnp.float32)],
            [pltpu.SemaphoreType.DMA, pltpu.SemaphoreType.DMA],
            [pltpu.SemaphoreType.DMA, pltpu.SemaphoreType.DMA],
        ],
    )
    return fn(idx, adjf, tab)


def kernel(adj_matrix, dep_rel_matrix, dep_table):
    idx = dep_rel_matrix.reshape(-1).astype(jnp.int32)
    adjf = adj_matrix.reshape(-1).astype(jnp.float32)
    tab = dep_table.reshape(-1).astype(jnp.float32)
    out = _sc_call(idx, adjf, tab)
    return out.transpose(0, 1, 3, 2)


# speedup vs baseline: 5.5137x; 1.0279x over previous
import jax
import jax.numpy as jnp
from jax import lax
from jax.experimental import pallas as pl
from jax.experimental.pallas import tpu as pltpu
from jax.experimental.pallas import tpu_sc as plsc

DEP_VOCAB = 50
EMBED_DIM = 64
B, S = 8, 256
N = B * S * S
NC, NS = 2, 16
NW = NC * NS
NROWS = B * S
ROWS_PER_W = NROWS // NW
IN_ROWS = 8
OUT_ROWS = 2
SUBS = IN_ROWS // OUT_ROWS
NIN = ROWS_PER_W // IN_ROWS
NOUT = ROWS_PER_W // OUT_ROWS
LANES = 16


def _sc_body(idx_hbm, adj_hbm, tab_hbm, out_hbm, tab_v, idx_vs, adj_vs,
             out_vs, in_sems, out_sems):
    wid = lax.axis_index("s") * NC + lax.axis_index("c")
    row0 = wid * ROWS_PER_W
    pltpu.sync_copy(tab_hbm, tab_v)

    def in_descs(k, buf):
        rbase = jnp.minimum(row0 + k * IN_ROWS, NROWS - IN_ROWS)
        bb = rbase >> 8
        ii = pl.multiple_of(rbase & (S - 1), IN_ROWS)
        return (
            pltpu.make_async_copy(idx_hbm.at[bb, pl.ds(ii, IN_ROWS)],
                                  idx_vs[buf], in_sems[buf]),
            pltpu.make_async_copy(adj_hbm.at[bb, pl.ds(ii, IN_ROWS)],
                                  adj_vs[buf], in_sems[buf]),
        )

    def out_desc(ci, obuf):
        rbase = row0 + ci * OUT_ROWS
        return pltpu.make_async_copy(
            out_vs[obuf],
            out_hbm.at[rbase >> 8, pl.ds(rbase & (S - 1), OUT_ROWS)],
            out_sems[obuf])

    def compute(buf, sub, obuf):
        ivb = idx_vs[buf]
        avb = adj_vs[buf]
        ovb = out_vs[obuf]

        def group_body(g, c2):
            r0 = g >> 4
            j0 = (g & 15) << 4
            idx16 = ivb[sub * OUT_ROWS + r0, pl.ds(j0, LANES)]
            adj16 = avb[sub * OUT_ROWS + r0, pl.ds(j0, LANES)]
            idx64 = idx16 * EMBED_DIM
            iota = lax.iota(jnp.int32, LANES)
            rvec = jnp.full((LANES,), r0, dtype=jnp.int32)
            jvec = iota + j0

            @plsc.parallel_loop(0, LANES, unroll=4)
            def s_loop(s):
                ps = (iota + s) & (LANES - 1)
                idxp = idx64 + ps
                for c in range(4):
                    r = plsc.load_gather(tab_v, [idxp + c * LANES])
                    plsc.store_scatter(ovb, [rvec, ps + c * LANES, jvec],
                                       r * adj16)

            return c2

        lax.fori_loop(0, OUT_ROWS * S // LANES, group_body, 0)

    for d in in_descs(0, 0) + in_descs(1, 1):
        d.start()

    def super_body(sc, carry):
        for buf in range(2):
            k = sc * 2 + buf
            for d in in_descs(k, buf):
                d.wait()
            for sub in range(SUBS):
                ci = k * SUBS + sub
                obuf = sub & 1

                if buf * SUBS + sub >= 2:
                    out_desc(ci - 2, obuf).wait()
                else:
                    @pl.when(sc >= 1)
                    def _():
                        out_desc(ci - 2, obuf).wait()

                compute(buf, sub, obuf)
                out_desc(ci, obuf).start()
            for d in in_descs(k + 2, buf):
                d.start()
        return carry

    lax.fori_loop(0, NIN // 2, super_body, 0)
    out_desc(NOUT - 2, 0).wait()
    out_desc(NOUT - 1, 1).wait()
    for buf in range(2):
        for d in in_descs(NIN + buf, buf):
            d.wait()


@jax.jit
def _sc_call(idx, adjf, tab):
    mesh = plsc.VectorSubcoreMesh(core_axis_name="c", subcore_axis_name="s",
                                  num_cores=NC, num_subcores=NS)
    fn = pl.kernel(
        _sc_body,
        out_type=jax.ShapeDtypeStruct((B, S, EMBED_DIM, S), jnp.float32),
        mesh=mesh,
        compiler_params=pltpu.CompilerParams(needs_layout_passes=False),
        scratch_types=[
            pltpu.VMEM((DEP_VOCAB * EMBED_DIM,), jnp.float32),
            [pltpu.VMEM((IN_ROWS, S), jnp.int32),
             pltpu.VMEM((IN_ROWS, S), jnp.int32)],
            [pltpu.VMEM((IN_ROWS, S), jnp.float32),
             pltpu.VMEM((IN_ROWS, S), jnp.float32)],
            [pltpu.VMEM((OUT_ROWS, EMBED_DIM, S), jnp.float32),
             pltpu.VMEM((OUT_ROWS, EMBED_DIM, S), jnp.float32)],
            [pltpu.SemaphoreType.DMA, pltpu.SemaphoreType.DMA],
            [pltpu.SemaphoreType.DMA, pltpu.SemaphoreType.DMA],
        ],
    )
    return fn(idx, adjf, tab)


def kernel(adj_matrix, dep_rel_matrix, dep_table):
    idx = dep_rel_matrix.astype(jnp.int32)
    adjf = adj_matrix.astype(jnp.float32)
    tab = dep_table.reshape(-1).astype(jnp.float32)
    out = _sc_call(idx, adjf, tab)
    return out.transpose(0, 1, 3, 2)
